# Initial kernel scaffold; baseline (speedup 1.0000x reference)
import jax
import jax.numpy as jnp
from jax.experimental import pallas as pl


def kernel(x, ptrs, csr):
    raise NotImplementedError("write your pallas kernel here")


# speedup vs baseline: 187.9214x; 187.9214x over previous
import functools

import jax
import jax.numpy as jnp
from jax import lax
from jax.experimental import pallas as pl
from jax.experimental.pallas import tpu as pltpu
from jax.experimental.pallas import tpu_sc as plsc

NC, NS = 2, 16
WINE = 16384


def _sc_segsum(x, ptrs, csr):
    n_nodes = x.shape[0]
    n_edges = ptrs.shape[0]
    per_tile = n_edges // (NC * NS)
    nwin = per_tile // WINE
    tail = per_tile - nwin * WINE
    slc = 6256
    last = n_nodes - (NS - 1) * slc

    mesh = plsc.VectorSubcoreMesh(core_axis_name="c", subcore_axis_name="s")

    @functools.partial(
        pl.kernel,
        out_type=jax.ShapeDtypeStruct((NC * n_nodes,), jnp.float32),
        mesh=mesh,
        scratch_types=[
            pltpu.VMEM_SHARED((n_nodes,), jnp.float32),
            pltpu.VMEM_SHARED((n_nodes,), jnp.float32),
            pltpu.VMEM((WINE,), jnp.int32),
            pltpu.VMEM((WINE,), jnp.int32),
            pltpu.VMEM((WINE,), jnp.float32),
            pltpu.VMEM((tail,), jnp.int32),
            pltpu.VMEM((tail,), jnp.int32),
            pltpu.VMEM((tail,), jnp.float32),
            pltpu.SemaphoreType.DMA,
        ],
    )
    def k(x_hbm, ptrs_hbm, csr_hbm, out_hbm,
          x_sh, acc_sh, ptr_v, csr_v, val_v, tptr_v, tcsr_v, tval_v, sem):
        c = lax.axis_index("c")
        s = lax.axis_index("s")

        def stage(off, size):
            buf = val_v.at[pl.ds(0, size)]
            pltpu.sync_copy(x_hbm.at[pl.ds(off, size)], buf)
            pltpu.sync_copy(buf, x_sh.at[pl.ds(off, size)])

            def zfill(i, carry):
                val_v[pl.ds(i * 16, 16)] = jnp.zeros((16,), jnp.float32)
                return carry

            lax.fori_loop(0, size // 16, zfill, 0)
            pltpu.sync_copy(buf, acc_sh.at[pl.ds(off, size)])

        @pl.when(s < NS - 1)
        def _():
            stage(s * slc, slc)

        @pl.when(s == NS - 1)
        def _():
            stage((NS - 1) * slc, last)

        plsc.subcore_barrier()

        base_e = (c * NS + s) * per_tile

        def window(w, carry):
            e0 = base_e + w * WINE
            pltpu.sync_copy(ptrs_hbm.at[pl.ds(e0, WINE)], ptr_v)
            pltpu.sync_copy(csr_hbm.at[pl.ds(e0, WINE)], csr_v)
            pltpu.async_copy(x_sh.at[ptr_v], val_v, sem).wait()
            pltpu.sync_copy(val_v, acc_sh.at[csr_v], add=True)
            return carry

        lax.fori_loop(0, nwin, window, 0)

        e0 = base_e + nwin * WINE
        pltpu.sync_copy(ptrs_hbm.at[pl.ds(e0, tail)], tptr_v)
        pltpu.sync_copy(csr_hbm.at[pl.ds(e0, tail)], tcsr_v)
        pltpu.async_copy(x_sh.at[tptr_v], tval_v, sem).wait()
        pltpu.sync_copy(tval_v, acc_sh.at[tcsr_v], add=True)

        plsc.subcore_barrier()

        def writeback(off, size):
            buf = val_v.at[pl.ds(0, size)]
            pltpu.sync_copy(acc_sh.at[pl.ds(off, size)], buf)
            pltpu.sync_copy(buf, out_hbm.at[pl.ds(c * n_nodes + off, size)])

        @pl.when(s < NS - 1)
        def _():
            writeback(s * slc, slc)

        @pl.when(s == NS - 1)
        def _():
            writeback((NS - 1) * slc, last)

    return k(x, ptrs, csr)


def _combine(p_ref, o_ref):
    o_ref[...] = p_ref[0, :] + p_ref[1, :]


def kernel(x, ptrs, csr):
    n_nodes = x.shape[0]
    p1 = ptrs.astype(jnp.int32)
    c1 = csr.astype(jnp.int32)
    partials = _sc_segsum(x.astype(jnp.float32), p1, c1)
    out = pl.pallas_call(
        _combine,
        out_shape=jax.ShapeDtypeStruct((n_nodes,), jnp.float32),
    )(partials.reshape(NC, n_nodes))
    return out


# speedup vs baseline: 223.5918x; 1.1898x over previous
import functools

import jax
import jax.numpy as jnp
from jax import lax
from jax.experimental import pallas as pl
from jax.experimental.pallas import tpu as pltpu
from jax.experimental.pallas import tpu_sc as plsc

NC, NS = 2, 16
WINE = 5000
NSLOT = 4


def _sc_segsum(x, ptrs, csr):
    n_nodes = x.shape[0]
    n_edges = ptrs.shape[0]
    per_tile = n_edges // (NC * NS)
    nwin = per_tile // WINE
    ngrp = nwin // NSLOT
    assert per_tile % WINE == 0 and nwin % NSLOT == 0
    slc = 6256
    last = n_nodes - (NS - 1) * slc

    mesh = plsc.VectorSubcoreMesh(core_axis_name="c", subcore_axis_name="s")

    @functools.partial(
        pl.kernel,
        out_type=jax.ShapeDtypeStruct((NC * n_nodes,), jnp.float32),
        mesh=mesh,
        scratch_types=[
            pltpu.VMEM_SHARED((n_nodes,), jnp.float32),
            pltpu.VMEM_SHARED((n_nodes,), jnp.float32),
            [pltpu.VMEM((WINE,), jnp.int32)] * NSLOT,
            [pltpu.VMEM((WINE,), jnp.int32)] * NSLOT,
            [pltpu.VMEM((WINE,), jnp.float32)] * NSLOT,
            pltpu.VMEM((slc,), jnp.float32),
            [pltpu.SemaphoreType.DMA] * NSLOT,
            [pltpu.SemaphoreType.DMA] * NSLOT,
            [pltpu.SemaphoreType.DMA] * NSLOT,
        ],
    )
    def k(x_hbm, ptrs_hbm, csr_hbm, out_hbm,
          x_sh, acc_sh, ptr_v, csr_v, val_v, stage_v, semL, semG, semS):
        c = lax.axis_index("c")
        s = lax.axis_index("s")

        def stage(off, size):
            buf = stage_v.at[pl.ds(0, size)]
            pltpu.sync_copy(x_hbm.at[pl.ds(off, size)], buf)
            pltpu.sync_copy(buf, x_sh.at[pl.ds(off, size)])

            def zfill(i, carry):
                stage_v[pl.ds(i * 16, 16)] = jnp.zeros((16,), jnp.float32)
                return carry

            lax.fori_loop(0, size // 16, zfill, 0)
            pltpu.sync_copy(buf, acc_sh.at[pl.ds(off, size)])

        @pl.when(s < NS - 1)
        def _():
            stage(s * slc, slc)

        @pl.when(s == NS - 1)
        def _():
            stage((NS - 1) * slc, last)

        plsc.subcore_barrier()

        base_e = (c * NS + s) * per_tile

        def group(g, carry):
            e0 = base_e + g * (NSLOT * WINE)
            dl = []
            for k in range(NSLOT):
                dl.append((
                    pltpu.async_copy(
                        ptrs_hbm.at[pl.ds(e0 + k * WINE, WINE)],
                        ptr_v[k], semL[k]),
                    pltpu.async_copy(
                        csr_hbm.at[pl.ds(e0 + k * WINE, WINE)],
                        csr_v[k], semL[k]),
                ))
            dg = []
            for k in range(NSLOT):
                dl[k][0].wait()
                dl[k][1].wait()
                dg.append(pltpu.async_copy(
                    x_sh.at[ptr_v[k]], val_v[k], semG[k]))
            ds = []
            for k in range(NSLOT):
                dg[k].wait()
                ds.append(pltpu.async_copy(
                    val_v[k], acc_sh.at[csr_v[k]], semS[k], add=True))
            for k in range(NSLOT):
                ds[k].wait()
            return carry

        lax.fori_loop(0, ngrp, group, 0)

        plsc.subcore_barrier()

        def writeback(off, size):
            buf = stage_v.at[pl.ds(0, size)]
            pltpu.sync_copy(acc_sh.at[pl.ds(off, size)], buf)
            pltpu.sync_copy(buf, out_hbm.at[pl.ds(c * n_nodes + off, size)])

        @pl.when(s < NS - 1)
        def _():
            writeback(s * slc, slc)

        @pl.when(s == NS - 1)
        def _():
            writeback((NS - 1) * slc, last)

    return k(x, ptrs, csr)


def _combine(p_ref, o_ref):
    o_ref[...] = p_ref[0, :] + p_ref[1, :]


def kernel(x, ptrs, csr):
    n_nodes = x.shape[0]
    p1 = ptrs.astype(jnp.int32)
    c1 = csr.astype(jnp.int32)
    partials = _sc_segsum(x.astype(jnp.float32), p1, c1)
    out = pl.pallas_call(
        _combine,
        out_shape=jax.ShapeDtypeStruct((n_nodes,), jnp.float32),
    )(partials.reshape(NC, n_nodes))
    return out
